# baseline (speedup 1.0000x reference)
import functools

import jax
import jax.numpy as jnp
from jax.experimental import pallas as pl
from jax.experimental.pallas import tpu as pltpu

_S = 16


def _fused_kernel(xp_ref, w_ref, b_ref, o_ref, *, H, W):
    W2 = 2 * W
    C = w_ref.shape[1]
    t = pl.program_id(1)
    xu = xp_ref[0, :, pl.ds(t * _S * W2, (_S + 2) * W2)]

    nlane = _S * W2
    lane = jax.lax.broadcasted_iota(jnp.int32, (1, nlane), 1) % W2
    not_first_col = lane > 0
    not_last_col = lane < (W2 - 1)
    zcol = jnp.zeros((C, 1), jnp.float32)

    def from_left(z):
        s = jnp.concatenate([zcol, z[:, :nlane - 1]], axis=1)
        return jnp.where(not_first_col, s, 0.0)

    def from_right(z):
        s = jnp.concatenate([z[:, 1:], zcol], axis=1)
        return jnp.where(not_last_col, s, 0.0)

    def col_tap(dx):
        z0 = jnp.dot(w_ref[0 * 3 + dx], xu, preferred_element_type=jnp.float32)
        z1 = jnp.dot(w_ref[1 * 3 + dx], xu, preferred_element_type=jnp.float32)
        z2 = jnp.dot(w_ref[2 * 3 + dx], xu, preferred_element_type=jnp.float32)
        t12 = z1 + z2
        t01 = z0 + z1
        e = z0[:, :nlane] + t12[:, W2:W2 + nlane]
        o = t01[:, W2:W2 + nlane] + z2[:, 2 * W2:2 * W2 + nlane]
        return e, o

    bias = b_ref[...]
    e0, o0 = col_tap(0)
    ev = from_left(e0) + bias
    od = from_left(o0) + bias
    del e0, o0
    e1, o1 = col_tap(1)
    ev = ev + e1
    od = od + o1
    del e1, o1
    e2, o2 = col_tap(2)
    ev = ev + from_right(e2)
    od = od + from_right(o2)

    ev = ev.astype(o_ref.dtype)
    od = od.astype(o_ref.dtype)
    for j in range(_S):
        o_ref[0, :, pl.ds(2 * j * W2, W2)] = ev[:, j * W2:(j + 1) * W2]
        o_ref[0, :, pl.ds((2 * j + 1) * W2, W2)] = od[:, j * W2:(j + 1) * W2]


@jax.jit
def kernel(x, weight, bias):
    N, C, H, W = x.shape

    xp = jnp.pad(x.astype(jnp.bfloat16), ((0, 0), (0, 0), (1, 1), (0, 0)))
    xp = jnp.repeat(xp, 2, axis=3).reshape(N, C, (H + 2) * 2 * W)
    w9 = jnp.transpose(weight, (2, 3, 0, 1)).reshape(9, C, C)
    w9 = w9.astype(jnp.bfloat16)
    b2 = bias.reshape(C, 1).astype(jnp.float32)

    T = H // _S
    out = pl.pallas_call(
        functools.partial(_fused_kernel, H=H, W=W),
        out_shape=jax.ShapeDtypeStruct((N, C, 4 * H * W), x.dtype),
        grid=(N, T),
        in_specs=[
            pl.BlockSpec((1, C, (H + 2) * 2 * W), lambda n, t: (n, 0, 0)),
            pl.BlockSpec((9, C, C), lambda n, t: (0, 0, 0)),
            pl.BlockSpec((C, 1), lambda n, t: (0, 0)),
        ],
        out_specs=pl.BlockSpec((1, C, _S * 4 * W), lambda n, t: (n, 0, t)),
        compiler_params=pltpu.CompilerParams(
            dimension_semantics=("parallel", "arbitrary"),
            vmem_limit_bytes=64 * 1024 * 1024),
        cost_estimate=pl.CostEstimate(
            flops=2 * 9 * C * C * (_S + 2) * 2 * W * H // _S * N,
            transcendentals=0,
            bytes_accessed=(N * C * (H + 2) * W * 2 + 9 * C * C * 2 + C * 4
                            + N * C * 4 * H * W * 4)),
    )(xp, w9, b2)
    return out.reshape(N, C, 2 * H, 2 * W)


# speedup vs baseline: 1.1191x; 1.1191x over previous
import functools

import jax
import jax.numpy as jnp
from jax.experimental import pallas as pl
from jax.experimental.pallas import tpu as pltpu

_S = 16


def _fused_kernel(x_ref, w_ref, b_ref, r_ref, o_ref, *, H, W):
    W2 = 2 * W
    C = w_ref.shape[1]
    t = pl.program_id(1)
    T = H // _S
    base = t * (_S * W)

    xmain = x_ref[0, :, pl.ds(pl.multiple_of(base, 128), _S * W)]
    topw = x_ref[0, :, pl.ds(pl.multiple_of(jnp.maximum(base - 128, 0), 128),
                             128)]
    xtop = jnp.where(t == 0, 0.0, topw[:, 128 - W:])
    botw = x_ref[0, :, pl.ds(pl.multiple_of(
        jnp.minimum(base + _S * W, H * W - 128), 128), 128)]
    xbot = jnp.where(t == T - 1, 0.0, botw[:, :W])
    xs = jnp.concatenate([xtop, xmain, xbot], axis=1).astype(jnp.bfloat16)

    xu = jnp.dot(xs, r_ref[...],
                 preferred_element_type=jnp.float32).astype(jnp.bfloat16)

    n2 = _S * W2
    lane2 = jax.lax.broadcasted_iota(jnp.int32, (1, n2), 1) % W2
    not_first_col = lane2 > 0
    not_last_col = lane2 < (W2 - 1)
    zcol = jnp.zeros((C, 1), jnp.float32)

    def from_left(z):
        s = jnp.concatenate([zcol, z[:, :n2 - 1]], axis=1)
        return jnp.where(not_first_col, s, 0.0)

    def from_right(z):
        s = jnp.concatenate([z[:, 1:], zcol], axis=1)
        return jnp.where(not_last_col, s, 0.0)

    def col_tap(dx):
        z0 = jnp.dot(w_ref[0 * 3 + dx], xu, preferred_element_type=jnp.float32)
        z1 = jnp.dot(w_ref[1 * 3 + dx], xu, preferred_element_type=jnp.float32)
        z2 = jnp.dot(w_ref[2 * 3 + dx], xu, preferred_element_type=jnp.float32)
        t12 = z1 + z2
        t01 = z0 + z1
        e = z0[:, :n2] + t12[:, W2:W2 + n2]
        o = t01[:, W2:W2 + n2] + z2[:, 2 * W2:2 * W2 + n2]
        return e, o

    bias = b_ref[...]
    e0, o0 = col_tap(0)
    ev = from_left(e0) + bias
    od = from_left(o0) + bias
    del e0, o0
    e1, o1 = col_tap(1)
    ev = ev + e1
    od = od + o1
    del e1, o1
    e2, o2 = col_tap(2)
    ev = ev + from_right(e2)
    od = od + from_right(o2)

    ev = ev.astype(o_ref.dtype)
    od = od.astype(o_ref.dtype)
    for j in range(_S):
        o_ref[0, :, pl.ds(2 * j * W2, W2)] = ev[:, j * W2:(j + 1) * W2]
        o_ref[0, :, pl.ds((2 * j + 1) * W2, W2)] = od[:, j * W2:(j + 1) * W2]


@jax.jit
def kernel(x, weight, bias):
    N, C, H, W = x.shape

    x_flat = x.reshape(N, C, H * W)
    w9 = jnp.transpose(weight, (2, 3, 0, 1)).reshape(9, C, C)
    w9 = w9.astype(jnp.bfloat16)
    b2 = bias.reshape(C, 1).astype(jnp.float32)

    nh = (_S + 2) * W
    src = jax.lax.broadcasted_iota(jnp.int32, (nh, 2 * nh), 0)
    dst = jax.lax.broadcasted_iota(jnp.int32, (nh, 2 * nh), 1)
    rmat = (src == (dst // (2 * W)) * W + (dst % (2 * W)) // 2)
    rmat = rmat.astype(jnp.bfloat16)

    T = H // _S
    out = pl.pallas_call(
        functools.partial(_fused_kernel, H=H, W=W),
        out_shape=jax.ShapeDtypeStruct((N, C, 4 * H * W), x.dtype),
        grid=(N, T),
        in_specs=[
            pl.BlockSpec((1, C, H * W), lambda n, t: (n, 0, 0)),
            pl.BlockSpec((9, C, C), lambda n, t: (0, 0, 0)),
            pl.BlockSpec((C, 1), lambda n, t: (0, 0)),
            pl.BlockSpec((nh, 2 * nh), lambda n, t: (0, 0)),
        ],
        out_specs=pl.BlockSpec((1, C, _S * 4 * W), lambda n, t: (n, 0, t)),
        compiler_params=pltpu.CompilerParams(
            dimension_semantics=("parallel", "arbitrary"),
            vmem_limit_bytes=64 * 1024 * 1024),
        cost_estimate=pl.CostEstimate(
            flops=2 * 9 * C * C * (_S + 2) * 2 * W * H // _S * N,
            transcendentals=0,
            bytes_accessed=(N * C * H * W * 4 + 9 * C * C * 2 + C * 4
                            + N * C * 4 * H * W * 4)),
    )(x_flat, w9, b2, rmat)
    return out.reshape(N, C, 2 * H, 2 * W)


# speedup vs baseline: 2.7987x; 2.5007x over previous
import functools

import jax
import jax.numpy as jnp
from jax.experimental import pallas as pl
from jax.experimental.pallas import tpu as pltpu

_S = 16


def _fused_kernel(x_ref, we_ref, wo_ref, b_ref, o_ref, *, H, W):
    C = x_ref.shape[2]
    t = pl.program_id(1)
    T = H // _S
    n = _S * W
    base = t * n

    xmain = x_ref[0, pl.ds(pl.multiple_of(base, 8), n), :]
    topw = x_ref[0, pl.ds(pl.multiple_of(jnp.maximum(base - 64, 0), 8), 64), :]
    xtop = jnp.where(t == 0, 0.0, topw[64 - W - 1:, :])
    botw = x_ref[0, pl.ds(pl.multiple_of(
        jnp.minimum(base + n, H * W - 64), 8), 64), :]
    xbot = jnp.where(t == T - 1, 0.0, botw[:W + 1, :])
    xcat = jnp.concatenate([xtop, xmain, xbot], axis=0)
    xbf = xcat[1:1 + n + 2 * W, :].astype(jnp.bfloat16)

    xa = xbf[0:n, :]
    xc = xbf[W:W + n, :]
    xb = xbf[2 * W:2 * W + n, :]
    xe2 = jnp.concatenate([xa, xc], axis=1)
    xo2 = jnp.concatenate([xc, xb], axis=1)

    def dot(a, w):
        return jax.lax.dot_general(a, w, (((1,), (0,)), ((), ())),
                                   preferred_element_type=jnp.float32)

    e0 = dot(xe2, we_ref[0])
    e1 = dot(xe2, we_ref[1])
    e2 = dot(xe2, we_ref[2])
    o0 = dot(xo2, wo_ref[0])
    o1 = dot(xo2, wo_ref[1])
    o2 = dot(xo2, wo_ref[2])

    row = jax.lax.broadcasted_iota(jnp.int32, (n, 1), 0) % W
    not_first = row > 0
    not_last = row < (W - 1)
    zrow = jnp.zeros((1, C), jnp.float32)

    def from_left(z):
        s = jnp.concatenate([zrow, z[:n - 1, :]], axis=0)
        return jnp.where(not_first, s, 0.0)

    def from_right(z):
        s = jnp.concatenate([z[1:, :], zrow], axis=0)
        return jnp.where(not_last, s, 0.0)

    bias = b_ref[...]
    s12e = (e1 + bias) + e2
    s01e = e0 + (e1 + bias)
    p00 = from_left(e0) + s12e
    p01 = s01e + from_right(e2)
    s12o = (o1 + bias) + o2
    s01o = o0 + (o1 + bias)
    p10 = from_left(o0) + s12o
    p11 = s01o + from_right(o2)

    ev = jnp.stack([p00, p01], axis=1).reshape(_S, 2 * W, C)
    od = jnp.stack([p10, p11], axis=1).reshape(_S, 2 * W, C)
    out = jnp.stack([ev, od], axis=1).reshape(_S * 4 * W, C)
    o_ref[0] = out.astype(o_ref.dtype)


@jax.jit
def kernel(x, weight, bias):
    N, C, H, W = x.shape

    xT = jnp.transpose(x.reshape(N, C, H * W), (0, 2, 1))

    w9 = jnp.transpose(weight, (2, 3, 1, 0)).reshape(3, 3, C, C)
    we = jnp.stack([jnp.concatenate([w9[0, dx], w9[1, dx] + w9[2, dx]], axis=0)
                    for dx in range(3)]).astype(jnp.bfloat16)
    wo = jnp.stack([jnp.concatenate([w9[0, dx] + w9[1, dx], w9[2, dx]], axis=0)
                    for dx in range(3)]).astype(jnp.bfloat16)
    b2 = bias.reshape(1, C).astype(jnp.float32)

    T = H // _S
    out = pl.pallas_call(
        functools.partial(_fused_kernel, H=H, W=W),
        out_shape=jax.ShapeDtypeStruct((N, 4 * H * W, C), x.dtype),
        grid=(N, T),
        in_specs=[
            pl.BlockSpec((1, H * W, C), lambda n, t: (n, 0, 0)),
            pl.BlockSpec((3, 2 * C, C), lambda n, t: (0, 0, 0)),
            pl.BlockSpec((3, 2 * C, C), lambda n, t: (0, 0, 0)),
            pl.BlockSpec((1, C), lambda n, t: (0, 0)),
        ],
        out_specs=pl.BlockSpec((1, _S * 4 * W, C), lambda n, t: (n, t, 0)),
        compiler_params=pltpu.CompilerParams(
            dimension_semantics=("parallel", "arbitrary"),
            vmem_limit_bytes=64 * 1024 * 1024),
        cost_estimate=pl.CostEstimate(
            flops=2 * 12 * C * C * H * W * N,
            transcendentals=0,
            bytes_accessed=(N * C * H * W * 4 + 12 * C * C * 2 + C * 4
                            + N * C * 4 * H * W * 4)),
    )(xT, we, wo, b2)
    return jnp.transpose(out, (0, 2, 1)).reshape(N, C, 2 * H, 2 * W)


# speedup vs baseline: 2.8883x; 1.0320x over previous
import functools

import jax
import jax.numpy as jnp
from jax.experimental import pallas as pl
from jax.experimental.pallas import tpu as pltpu

_S = 16


def _fused_kernel(x_ref, we_ref, wo_ref, b_ref, o_ref, *, H, W):
    C = x_ref.shape[2]
    t = pl.program_id(1)
    T = H // _S
    n = _S * W
    nh = n + 2 * W
    base = t * n

    xmain = x_ref[0, pl.ds(pl.multiple_of(base, 8), n), :]
    topw = x_ref[0, pl.ds(pl.multiple_of(jnp.maximum(base - 64, 0), 8), 64), :]
    xtop = jnp.where(t == 0, 0.0, topw[64 - W - 1:, :])
    botw = x_ref[0, pl.ds(pl.multiple_of(
        jnp.minimum(base + n, H * W - 64), 8), 64), :]
    xbot = jnp.where(t == T - 1, 0.0, botw[:W + 1, :])
    xcat = jnp.concatenate([xtop, xmain, xbot], axis=0)
    xbf = xcat[1:1 + n + 2 * W, :].astype(jnp.bfloat16)

    roww = jax.lax.broadcasted_iota(jnp.int32, (nh, 1), 0) % W
    zrow = jnp.zeros((1, C), jnp.bfloat16)
    xlf = jnp.where(roww > 0,
                    jnp.concatenate([zrow, xbf[:nh - 1, :]], axis=0),
                    jnp.bfloat16(0))
    xrt = jnp.where(roww < (W - 1),
                    jnp.concatenate([xbf[1:, :], zrow], axis=0),
                    jnp.bfloat16(0))

    def quad(xs, o1_, o2_):
        return jnp.concatenate([xs[o1_:o1_ + n, :], xs[o2_:o2_ + n, :]],
                               axis=1)

    xe_l = quad(xlf, 0, W)
    xe_c = quad(xbf, 0, W)
    xe_r = quad(xrt, 0, W)
    xo_l = quad(xlf, W, 2 * W)
    xo_c = quad(xbf, W, 2 * W)
    xo_r = quad(xrt, W, 2 * W)

    def dot(a, w):
        return jax.lax.dot_general(a, w, (((1,), (0,)), ((), ())),
                                   preferred_element_type=jnp.float32)

    bias = b_ref[...]
    p00 = dot(jnp.concatenate([xe_l, xe_c], axis=1), we_ref[0]) + bias
    p01 = dot(jnp.concatenate([xe_c, xe_r], axis=1), we_ref[1]) + bias
    p10 = dot(jnp.concatenate([xo_l, xo_c], axis=1), wo_ref[0]) + bias
    p11 = dot(jnp.concatenate([xo_c, xo_r], axis=1), wo_ref[1]) + bias

    ev = jnp.stack([p00, p01], axis=1).reshape(_S, 2 * W, C)
    od = jnp.stack([p10, p11], axis=1).reshape(_S, 2 * W, C)
    out = jnp.stack([ev, od], axis=1).reshape(_S * 4 * W, C)
    o_ref[0] = out.astype(o_ref.dtype)


@jax.jit
def kernel(x, weight, bias):
    N, C, H, W = x.shape

    xT = jnp.transpose(x.reshape(N, C, H * W), (0, 2, 1))

    w9 = jnp.transpose(weight, (2, 3, 1, 0)).reshape(3, 3, C, C)
    wef = [jnp.concatenate([w9[0, dx], w9[1, dx] + w9[2, dx]], axis=0)
           for dx in range(3)]
    wof = [jnp.concatenate([w9[0, dx] + w9[1, dx], w9[2, dx]], axis=0)
           for dx in range(3)]
    we = jnp.stack([jnp.concatenate([wef[0], wef[1] + wef[2]], axis=0),
                    jnp.concatenate([wef[0] + wef[1], wef[2]], axis=0)])
    wo = jnp.stack([jnp.concatenate([wof[0], wof[1] + wof[2]], axis=0),
                    jnp.concatenate([wof[0] + wof[1], wof[2]], axis=0)])
    we = we.astype(jnp.bfloat16)
    wo = wo.astype(jnp.bfloat16)
    b2 = bias.reshape(1, C).astype(jnp.float32)

    T = H // _S
    out = pl.pallas_call(
        functools.partial(_fused_kernel, H=H, W=W),
        out_shape=jax.ShapeDtypeStruct((N, 4 * H * W, C), x.dtype),
        grid=(N, T),
        in_specs=[
            pl.BlockSpec((1, H * W, C), lambda n, t: (n, 0, 0)),
            pl.BlockSpec((2, 4 * C, C), lambda n, t: (0, 0, 0)),
            pl.BlockSpec((2, 4 * C, C), lambda n, t: (0, 0, 0)),
            pl.BlockSpec((1, C), lambda n, t: (0, 0)),
        ],
        out_specs=pl.BlockSpec((1, _S * 4 * W, C), lambda n, t: (n, t, 0)),
        compiler_params=pltpu.CompilerParams(
            dimension_semantics=("parallel", "arbitrary"),
            vmem_limit_bytes=64 * 1024 * 1024),
        cost_estimate=pl.CostEstimate(
            flops=2 * 12 * C * C * H * W * N,
            transcendentals=0,
            bytes_accessed=(N * C * H * W * 4 + 12 * C * C * 2 + C * 4
                            + N * C * 4 * H * W * 4)),
    )(xT, we, wo, b2)
    return jnp.transpose(out, (0, 2, 1)).reshape(N, C, 2 * H, 2 * W)


# speedup vs baseline: 3.2662x; 1.1309x over previous
import functools

import jax
import jax.numpy as jnp
from jax.experimental import pallas as pl
from jax.experimental.pallas import tpu as pltpu

_S = 32


def _fused_kernel(x_ref, we_ref, wo_ref, b_ref, o_ref, *, H, W):
    C = x_ref.shape[2]
    t = pl.program_id(1)
    T = H // _S
    n = _S * W
    nh = n + 2 * W
    base = t * n

    xmain = x_ref[0, pl.ds(pl.multiple_of(base, 8), n), :]
    topw = x_ref[0, pl.ds(pl.multiple_of(jnp.maximum(base - 64, 0), 8), 64), :]
    xtop = jnp.where(t == 0, 0.0, topw[64 - W - 1:, :])
    botw = x_ref[0, pl.ds(pl.multiple_of(
        jnp.minimum(base + n, H * W - 64), 8), 64), :]
    xbot = jnp.where(t == T - 1, 0.0, botw[:W + 1, :])
    xcat = jnp.concatenate([xtop, xmain, xbot], axis=0)
    xbf = xcat[1:1 + n + 2 * W, :].astype(jnp.bfloat16)

    roww = jax.lax.broadcasted_iota(jnp.int32, (nh, 1), 0) % W
    zrow = jnp.zeros((1, C), jnp.bfloat16)
    xlf = jnp.where(roww > 0,
                    jnp.concatenate([zrow, xbf[:nh - 1, :]], axis=0),
                    jnp.bfloat16(0))
    xrt = jnp.where(roww < (W - 1),
                    jnp.concatenate([xbf[1:, :], zrow], axis=0),
                    jnp.bfloat16(0))

    def quad(xs, o1_, o2_):
        return jnp.concatenate([xs[o1_:o1_ + n, :], xs[o2_:o2_ + n, :]],
                               axis=1)

    xe_l = quad(xlf, 0, W)
    xe_c = quad(xbf, 0, W)
    xe_r = quad(xrt, 0, W)
    xo_l = quad(xlf, W, 2 * W)
    xo_c = quad(xbf, W, 2 * W)
    xo_r = quad(xrt, W, 2 * W)

    def dot(a, w):
        return jax.lax.dot_general(a, w, (((1,), (0,)), ((), ())),
                                   preferred_element_type=jnp.float32)

    bias = b_ref[...]
    p00 = dot(jnp.concatenate([xe_l, xe_c], axis=1), we_ref[0]) + bias
    p01 = dot(jnp.concatenate([xe_c, xe_r], axis=1), we_ref[1]) + bias
    p10 = dot(jnp.concatenate([xo_l, xo_c], axis=1), wo_ref[0]) + bias
    p11 = dot(jnp.concatenate([xo_c, xo_r], axis=1), wo_ref[1]) + bias

    ev = jnp.stack([p00, p01], axis=1).reshape(_S, 2 * W, C)
    od = jnp.stack([p10, p11], axis=1).reshape(_S, 2 * W, C)
    out = jnp.stack([ev, od], axis=1).reshape(_S * 4 * W, C)
    o_ref[0] = out.astype(o_ref.dtype)


@jax.jit
def kernel(x, weight, bias):
    N, C, H, W = x.shape

    xT = jnp.transpose(x.reshape(N, C, H * W), (0, 2, 1))

    w9 = jnp.transpose(weight, (2, 3, 1, 0)).reshape(3, 3, C, C)
    wef = [jnp.concatenate([w9[0, dx], w9[1, dx] + w9[2, dx]], axis=0)
           for dx in range(3)]
    wof = [jnp.concatenate([w9[0, dx] + w9[1, dx], w9[2, dx]], axis=0)
           for dx in range(3)]
    we = jnp.stack([jnp.concatenate([wef[0], wef[1] + wef[2]], axis=0),
                    jnp.concatenate([wef[0] + wef[1], wef[2]], axis=0)])
    wo = jnp.stack([jnp.concatenate([wof[0], wof[1] + wof[2]], axis=0),
                    jnp.concatenate([wof[0] + wof[1], wof[2]], axis=0)])
    we = we.astype(jnp.bfloat16)
    wo = wo.astype(jnp.bfloat16)
    b2 = bias.reshape(1, C).astype(jnp.float32)

    T = H // _S
    out = pl.pallas_call(
        functools.partial(_fused_kernel, H=H, W=W),
        out_shape=jax.ShapeDtypeStruct((N, 4 * H * W, C), x.dtype),
        grid=(N, T),
        in_specs=[
            pl.BlockSpec((1, H * W, C), lambda n, t: (n, 0, 0)),
            pl.BlockSpec((2, 4 * C, C), lambda n, t: (0, 0, 0)),
            pl.BlockSpec((2, 4 * C, C), lambda n, t: (0, 0, 0)),
            pl.BlockSpec((1, C), lambda n, t: (0, 0)),
        ],
        out_specs=pl.BlockSpec((1, _S * 4 * W, C), lambda n, t: (n, t, 0)),
        compiler_params=pltpu.CompilerParams(
            dimension_semantics=("parallel", "arbitrary"),
            vmem_limit_bytes=64 * 1024 * 1024),
        cost_estimate=pl.CostEstimate(
            flops=2 * 12 * C * C * H * W * N,
            transcendentals=0,
            bytes_accessed=(N * C * H * W * 4 + 12 * C * C * 2 + C * 4
                            + N * C * 4 * H * W * 4)),
    )(xT, we, wo, b2)
    return jnp.transpose(out, (0, 2, 1)).reshape(N, C, 2 * H, 2 * W)
